# baseline (speedup 1.0000x reference)
import jax
import jax.numpy as jnp
from jax.experimental import pallas as pl
from jax.experimental.pallas import tpu as pltpu

_B, _N, _M = 2, 4096, 3
_CHUNK = 2048


def _chamfer_body(x1_ref, x2t_ref, out_ref):
    B, N, _ = x1_ref.shape
    M = x2t_ref.shape[2]
    n_chunks = N // _CHUNK

    total = jnp.float32(0.0)
    for b in range(B):
        G = x2t_ref[b]
        r2 = jnp.sum(G * G, axis=0, keepdims=True)
        Ga = jnp.concatenate([G, r2], axis=0)

        def chunk_step(i, carry):
            sum1, min2 = carry
            q = x1_ref[b, pl.ds(i * _CHUNK, _CHUNK), :]
            q2 = jnp.sum(q * q, axis=1, keepdims=True)
            qa = jnp.concatenate(
                [-2.0 * q, jnp.ones((_CHUNK, 1), jnp.float32)], axis=1
            )
            dp = jax.lax.dot_general(
                qa, Ga, (((1,), (0,)), ((), ())),
                preferred_element_type=jnp.float32, precision=jax.lax.Precision.HIGHEST,
            )
            rmin = jnp.min(dp[0:8], axis=1, keepdims=True) + q2[0:8]
            sum1 = sum1 + jnp.sum(jnp.maximum(rmin, 0.0))
            min2 = jnp.minimum(min2, jnp.sum(q2) + dp[0])
            return sum1, min2

        sum1, min2 = jax.lax.fori_loop(
            0, n_chunks, chunk_step,
            (jnp.float32(0.0), jnp.full((M,), jnp.inf, jnp.float32)),
        )
        total = total + sum1 / (B * N) + jnp.sum(jnp.maximum(min2, 0.0)) / (B * M)

    out_ref[0, 0] = total


def kernel(xyz1, xyz2):
    x2t = jnp.transpose(xyz2, (0, 2, 1))
    out = pl.pallas_call(
        _chamfer_body,
        out_shape=jax.ShapeDtypeStruct((1, 1), jnp.float32),
        out_specs=pl.BlockSpec(memory_space=pltpu.SMEM),
    )(xyz1, x2t)
    return out[0, 0]


# speedup vs baseline: 1.1583x; 1.1583x over previous
import jax
import jax.numpy as jnp
from jax import lax
from jax.experimental import pallas as pl
from jax.experimental.pallas import tpu as pltpu
from jax.experimental.pallas import tpu_sc as plsc

_NC, _NS, _L = 2, 16, 16
_S_SC = 512
_QPT = _S_SC // _NS
_CHUNK = 1792


def _tc_body(x1_ref, x2t_ref, sum1_ref, min2_ref):
    B, N, _ = x1_ref.shape
    M = x2t_ref.shape[2]
    n_chunks = (N - _S_SC) // _CHUNK

    total = jnp.float32(0.0)
    for b in range(B):
        G = x2t_ref[b]
        r2 = jnp.sum(G * G, axis=0, keepdims=True)
        Ga = jnp.concatenate([G, r2], axis=0)

        def chunk_step(i, carry):
            sum1, min2 = carry
            q = x1_ref[b, pl.ds(_S_SC + i * _CHUNK, _CHUNK), :]
            q2 = jnp.sum(q * q, axis=1, keepdims=True)
            qa = jnp.concatenate(
                [-2.0 * q, jnp.ones((_CHUNK, 1), jnp.float32)], axis=1
            )
            dp = jax.lax.dot_general(
                qa, Ga, (((1,), (0,)), ((), ())),
                preferred_element_type=jnp.float32,
            )
            rmin = jnp.min(dp, axis=1, keepdims=True) + q2
            sum1 = sum1 + jnp.sum(jnp.maximum(rmin, 0.0))
            min2 = jnp.minimum(min2, jnp.min(dp + q2, axis=0))
            return sum1, min2

        sum1, min2 = jax.lax.fori_loop(
            0, n_chunks, chunk_step,
            (jnp.float32(0.0), jnp.full((M,), jnp.inf, jnp.float32)),
        )
        total = total + sum1
        min2_ref[b, :] = min2

    sum1_ref[0, 0] = total


def _tc_main(xyz1, x2t):
    B, N, _ = xyz1.shape
    M = x2t.shape[2]
    return pl.pallas_call(
        _tc_body,
        out_shape=(
            jax.ShapeDtypeStruct((1, 1), jnp.float32),
            jax.ShapeDtypeStruct((B, M), jnp.float32),
        ),
        out_specs=(
            pl.BlockSpec(memory_space=pltpu.SMEM),
            pl.BlockSpec(memory_space=pltpu.VMEM),
        ),
    )(xyz1, x2t)


def _sc_body(q_hbm, x2t_hbm, min1_hbm, cmin_hbm,
             q_v, r_v, cmin_v, min1_v):
    b = lax.axis_index("c")
    s = lax.axis_index("s")
    M = x2t_hbm.shape[2]
    n_ch = M // _L

    pltpu.sync_copy(x2t_hbm.at[b], r_v)
    pltpu.sync_copy(q_hbm.at[b, s], q_v)

    inf16 = jnp.full((_L,), jnp.inf, jnp.float32)

    def init_step(j, _):
        cmin_v[pl.ds(j * _L, _L)] = inf16
        return 0

    lax.fori_loop(0, n_ch, init_step, 0)

    for g in range(_QPT // _L):
        qxv = q_v[0, pl.ds(g * _L, _L)]
        qyv = q_v[1, pl.ds(g * _L, _L)]
        qzv = q_v[2, pl.ds(g * _L, _L)]
        for u in range(_L):
            qx = qxv[u]
            qy = qyv[u]
            qz = qzv[u]

            def r_step(j, rmin):
                base = j * _L
                dx = r_v[0, pl.ds(base, _L)] - qx
                dy = r_v[1, pl.ds(base, _L)] - qy
                dz = r_v[2, pl.ds(base, _L)] - qz
                d = dx * dx + dy * dy + dz * dz
                cmin_v[pl.ds(base, _L)] = jnp.minimum(
                    cmin_v[pl.ds(base, _L)], d)
                return jnp.minimum(rmin, d)

            rmin = lax.fori_loop(0, n_ch, r_step, inf16, unroll=4)
            min1_v[pl.ds((g * _L + u) * _L, _L)] = rmin

    pltpu.sync_copy(min1_v, min1_hbm.at[b, s])
    pltpu.sync_copy(cmin_v, cmin_hbm.at[b, s])


def _sc_slice(q_sc, x2t):
    B = x2t.shape[0]
    M = x2t.shape[2]
    mesh = plsc.VectorSubcoreMesh(core_axis_name="c", subcore_axis_name="s")
    return pl.kernel(
        _sc_body,
        out_type=(
            jax.ShapeDtypeStruct((B, _NS, _QPT * _L), jnp.float32),
            jax.ShapeDtypeStruct((B, _NS, M), jnp.float32),
        ),
        mesh=mesh,
        scratch_types=[
            pltpu.VMEM((3, _QPT), jnp.float32),
            pltpu.VMEM((3, M), jnp.float32),
            pltpu.VMEM((M,), jnp.float32),
            pltpu.VMEM((_QPT * _L,), jnp.float32),
        ],
    )(q_sc, x2t)


def _merge_body(sum1_ref, min2_ref, sc_min1_ref, sc_cmin_ref, out_ref):
    B, _, M = sc_cmin_ref.shape
    N = M

    min1_sum = sum1_ref[0, 0] + jnp.sum(
        jnp.maximum(jnp.min(sc_min1_ref[...], axis=1), 0.0))

    min2_sum = jnp.float32(0.0)
    for b in range(B):
        m2 = min2_ref[b, :]
        for s in range(_NS):
            m2 = jnp.minimum(m2, sc_cmin_ref[b, s, :])
        min2_sum = min2_sum + jnp.sum(jnp.maximum(m2, 0.0))

    out_ref[0, 0] = min1_sum / (B * N) + min2_sum / (B * M)


def _merge(sum1, min2, sc_min1, sc_cmin):
    return pl.pallas_call(
        _merge_body,
        out_shape=jax.ShapeDtypeStruct((1, 1), jnp.float32),
        in_specs=[
            pl.BlockSpec(memory_space=pltpu.SMEM),
            pl.BlockSpec(memory_space=pltpu.VMEM),
            pl.BlockSpec(memory_space=pltpu.VMEM),
            pl.BlockSpec(memory_space=pltpu.VMEM),
        ],
        out_specs=pl.BlockSpec(memory_space=pltpu.SMEM),
    )(sum1, min2, sc_min1, sc_cmin)


def kernel(xyz1, xyz2):
    B, N, _ = xyz1.shape
    x2t = jnp.transpose(xyz2, (0, 2, 1))
    q_sc = jnp.transpose(
        xyz1[:, :_S_SC, :].reshape(B, _NS, _QPT, 3), (0, 1, 3, 2)
    )

    sc_min1, sc_cmin = _sc_slice(q_sc, x2t)
    sum1, min2 = _tc_main(xyz1, x2t)
    out = _merge(sum1, min2, sc_min1.reshape(-1, _L), sc_cmin)
    return out[0, 0]


# speedup vs baseline: 4.3361x; 3.7435x over previous
import jax
import jax.numpy as jnp
from jax.experimental import pallas as pl
from jax.experimental.pallas import tpu as pltpu

_B, _N, _M = 2, 4096, 3
_CHUNK = 2048


def _chamfer_body(x1_ref, x2t_ref, out_ref):
    B, N, _ = x1_ref.shape
    M = x2t_ref.shape[2]
    n_chunks = N // _CHUNK

    total = jnp.float32(0.0)
    for b in range(B):
        G = x2t_ref[b]
        r2 = jnp.sum(G * G, axis=0, keepdims=True)
        Ga = jnp.concatenate([G, r2], axis=0).astype(jnp.bfloat16)

        def chunk_step(i, carry):
            sum1, min2 = carry
            q = x1_ref[b, pl.ds(i * _CHUNK, _CHUNK), :]
            q2 = jnp.sum(q * q, axis=1, keepdims=True)
            qa = jnp.concatenate(
                [-2.0 * q, jnp.ones((_CHUNK, 1), jnp.float32)], axis=1
            ).astype(jnp.bfloat16)
            dp = jax.lax.dot_general(
                qa, Ga, (((1,), (0,)), ((), ())),
                preferred_element_type=jnp.float32,
            )
            rmin = jnp.min(dp, axis=1, keepdims=True) + q2
            sum1 = sum1 + jnp.sum(jnp.maximum(rmin, 0.0))
            min2 = jnp.minimum(min2, jnp.min(dp + q2, axis=0))
            return sum1, min2

        sum1, min2 = jax.lax.fori_loop(
            0, n_chunks, chunk_step,
            (jnp.float32(0.0), jnp.full((M,), jnp.inf, jnp.float32)),
        )
        total = total + sum1 / (B * N) + jnp.sum(jnp.maximum(min2, 0.0)) / (B * M)

    out_ref[0, 0] = total


def kernel(xyz1, xyz2):
    x2t = jnp.transpose(xyz2, (0, 2, 1))
    out = pl.pallas_call(
        _chamfer_body,
        out_shape=jax.ShapeDtypeStruct((1, 1), jnp.float32),
        out_specs=pl.BlockSpec(memory_space=pltpu.SMEM),
    )(xyz1, x2t)
    return out[0, 0]
